# baseline (speedup 1.0000x reference)
import numpy as np
import jax
import jax.numpy as jnp
from jax.experimental import pallas as pl
from jax.experimental.pallas import tpu as pltpu

_U_DECAY = 0.97
_X_DECAY = 0.97
_LN_EPS = 1e-5
_L1_EPS = 1e-12
_NEG = -1e30


def _ln(x):
    m = jnp.mean(x, axis=-1, keepdims=True)
    v = jnp.mean((x - m) ** 2, axis=-1, keepdims=True)
    return (x - m) * jax.lax.rsqrt(v + _LN_EPS)


def _body(emb_ref, dx_ref, dy_ref, e_ref, dm_ref, lones_ref, t0s_ref,
          out_ref, xu0, xu1, xu2, xu3, xu4, xu5, xu6, xu7, sc_ref, cc_ref):
    xu = (xu0, xu1, xu2, xu3, xu4, xu5, xu6, xu7)
    nb = len(xu)
    t, n = xu0.shape
    emb = emb_ref[...]

    for b in range(nb):
        xu[b][...] = jnp.maximum(
            jax.lax.dot_general(
                emb[b], dx_ref[...], (((1,), (1,)), ((), ())),
                preferred_element_type=jnp.float32),
            0.0,
        )

    sc_ref[:, 0:nb] = jnp.concatenate(
        [jnp.sum(xu[b][...], axis=1, keepdims=True) for b in range(nb)],
        axis=1,
    )

    def sstep(k, sig):
        base = pl.multiple_of(k * 8, 8)
        tile = sc_ref[pl.ds(base, 8), :]
        cs = []
        for j in range(8):
            xi = _X_DECAY * sig + tile[j:j + 1, :]
            cs.append(jnp.maximum(xi, _L1_EPS))
            sig = jnp.minimum(xi * (1.0 / _L1_EPS), 1.0)
        cc_ref[pl.ds(base, 8), :] = jnp.concatenate(cs, axis=0)
        return sig

    jax.lax.fori_loop(0, t // 8, sstep, jnp.zeros((1, 128), jnp.float32),
                      unroll=2)

    cl = jnp.log(cc_ref[...])
    lc = jnp.dot(lones_ref[...], cl,
                 preferred_element_type=jnp.float32)
    lcs_t = jnp.transpose(lc - cl)

    dm = dm_ref[...]
    for b in range(nb):
        u = xu[b][...]
        w0 = jnp.exp(t0s_ref[...] + lcs_t[b:b + 1, :] - lc[:, b:b + 1])
        rcp = 1.0 / cc_ref[:, b:b + 1]
        x = u * rcp + jnp.dot(w0, u, preferred_element_type=jnp.float32)
        xu[b][...] = x

        vl = _ln(emb[b])
        g = jax.lax.dot_general(
            x, x, (((1,), (1,)), ((), ())),
            preferred_element_type=jnp.float32,
        )
        a = jnp.dot(dm * g, vl, preferred_element_type=jnp.float32)
        y = jnp.maximum(
            jax.lax.dot_general(
                _ln(a), dy_ref[...], (((1,), (1,)), ((), ())),
                preferred_element_type=jnp.float32),
            0.0,
        ) * x
        out_ref[b] = _ln(
            jax.lax.dot_general(
                y, e_ref[...], (((1,), (1,)), ((), ())),
                preferred_element_type=jnp.float32)
        )


def kernel(embeddings, E, Dx, Dy):
    b, t, d = embeddings.shape
    n = E.shape[1]
    nb = 8

    i = np.arange(t)
    expo = np.maximum(i[:, None] - 1 - i[None, :], 0)
    dm = np.where(i[None, :] < i[:, None],
                  np.power(np.float64(_U_DECAY), expo), 0.0).astype(np.float32)
    lones = (i[None, :] <= i[:, None]).astype(np.float32)
    t0s = np.where(i[None, :] < i[:, None],
                   np.log(np.float64(_X_DECAY)) * (i[:, None] - i[None, :]),
                   _NEG).astype(np.float32)

    return pl.pallas_call(
        _body,
        grid=(b // nb,),
        in_specs=[
            pl.BlockSpec((nb, t, d), lambda c: (c, 0, 0)),
            pl.BlockSpec((n, d), lambda c: (0, 0)),
            pl.BlockSpec((n, d), lambda c: (0, 0)),
            pl.BlockSpec((d, n), lambda c: (0, 0)),
            pl.BlockSpec((t, t), lambda c: (0, 0)),
            pl.BlockSpec((t, t), lambda c: (0, 0)),
            pl.BlockSpec((t, t), lambda c: (0, 0)),
        ],
        out_specs=pl.BlockSpec((nb, t, d), lambda c: (c, 0, 0)),
        out_shape=jax.ShapeDtypeStruct((b, t, d), jnp.float32),
        scratch_shapes=[pltpu.VMEM((t, n), jnp.float32) for _ in range(nb)]
        + [pltpu.VMEM((t, 128), jnp.float32) for _ in range(2)],
        compiler_params=pltpu.CompilerParams(
            dimension_semantics=("arbitrary",),
            vmem_limit_bytes=48 * 1024 * 1024,
        ),
        name="bdh_fused",
    )(embeddings, Dx, Dy, E,
      jnp.asarray(dm), jnp.asarray(lones), jnp.asarray(t0s))


# speedup vs baseline: 1.0324x; 1.0324x over previous
import numpy as np
import jax
import jax.numpy as jnp
from jax.experimental import pallas as pl
from jax.experimental.pallas import tpu as pltpu

_U_DECAY = 0.97
_X_DECAY = 0.97
_LN_EPS = 1e-5
_L1_EPS = 1e-12
_NEG = -1e30


def _ln(x):
    m = jnp.mean(x, axis=-1, keepdims=True)
    v = jnp.mean((x - m) ** 2, axis=-1, keepdims=True)
    return (x - m) * jax.lax.rsqrt(v + _LN_EPS)


def _body(emb_ref, dx_ref, dy_ref, e_ref, dm_ref, lones_ref, t0s_ref,
          out_ref, xu_ref, sc_ref, cc_ref):
    nbt, n = xu_ref.shape
    nb, t, d = emb_ref.shape
    emb = emb_ref[...]

    xu_ref[...] = jnp.maximum(
        jax.lax.dot_general(
            emb.reshape(nbt, d), dx_ref[...], (((1,), (1,)), ((), ())),
            preferred_element_type=jnp.float32),
        0.0,
    )

    sc_ref[:, 0:nb] = jnp.concatenate(
        [jnp.sum(xu_ref[b * t:(b + 1) * t, :], axis=1, keepdims=True)
         for b in range(nb)],
        axis=1,
    )

    def sstep(k, sig):
        base = pl.multiple_of(k * 8, 8)
        tile = sc_ref[pl.ds(base, 8), :]
        cs = []
        for j in range(8):
            xi = _X_DECAY * sig + tile[j:j + 1, :]
            cs.append(jnp.maximum(xi, _L1_EPS))
            sig = jnp.minimum(xi * (1.0 / _L1_EPS), 1.0)
        cc_ref[pl.ds(base, 8), :] = jnp.concatenate(cs, axis=0)
        return sig

    jax.lax.fori_loop(0, t // 8, sstep, jnp.zeros((1, 128), jnp.float32),
                      unroll=2)

    cl = jnp.log(cc_ref[...])
    lc = jnp.dot(lones_ref[...], cl,
                 preferred_element_type=jnp.float32)
    lcs_t = jnp.transpose(lc - cl)

    dm = dm_ref[...]
    a_list = []
    for b in range(nb):
        u = xu_ref[b * t:(b + 1) * t, :]
        w0 = jnp.exp(t0s_ref[...] + lcs_t[b:b + 1, :] - lc[:, b:b + 1])
        rcp = 1.0 / cc_ref[:, b:b + 1]
        x = u * rcp + jnp.dot(w0, u, preferred_element_type=jnp.float32)
        xu_ref[b * t:(b + 1) * t, :] = x

        g = jax.lax.dot_general(
            x, x, (((1,), (1,)), ((), ())),
            preferred_element_type=jnp.float32,
        )
        vl = _ln(emb[b])
        a_list.append(
            jnp.dot(dm * g, vl, preferred_element_type=jnp.float32))

    a_all = jnp.concatenate(a_list, axis=0)
    y = jnp.maximum(
        jax.lax.dot_general(
            _ln(a_all), dy_ref[...], (((1,), (1,)), ((), ())),
            preferred_element_type=jnp.float32),
        0.0,
    ) * xu_ref[...]
    out_ref[...] = _ln(
        jax.lax.dot_general(
            y, e_ref[...], (((1,), (1,)), ((), ())),
            preferred_element_type=jnp.float32)
    ).reshape(nb, t, d)


def kernel(embeddings, E, Dx, Dy):
    b, t, d = embeddings.shape
    n = E.shape[1]
    nb = 8

    i = np.arange(t)
    expo = np.maximum(i[:, None] - 1 - i[None, :], 0)
    dm = np.where(i[None, :] < i[:, None],
                  np.power(np.float64(_U_DECAY), expo), 0.0).astype(np.float32)
    lones = (i[None, :] <= i[:, None]).astype(np.float32)
    t0s = np.where(i[None, :] < i[:, None],
                   np.log(np.float64(_X_DECAY)) * (i[:, None] - i[None, :]),
                   _NEG).astype(np.float32)

    return pl.pallas_call(
        _body,
        grid=(b // nb,),
        in_specs=[
            pl.BlockSpec((nb, t, d), lambda c: (c, 0, 0)),
            pl.BlockSpec((n, d), lambda c: (0, 0)),
            pl.BlockSpec((n, d), lambda c: (0, 0)),
            pl.BlockSpec((d, n), lambda c: (0, 0)),
            pl.BlockSpec((t, t), lambda c: (0, 0)),
            pl.BlockSpec((t, t), lambda c: (0, 0)),
            pl.BlockSpec((t, t), lambda c: (0, 0)),
        ],
        out_specs=pl.BlockSpec((nb, t, d), lambda c: (c, 0, 0)),
        out_shape=jax.ShapeDtypeStruct((b, t, d), jnp.float32),
        scratch_shapes=[pltpu.VMEM((nb * t, n), jnp.float32)]
        + [pltpu.VMEM((t, 128), jnp.float32) for _ in range(2)],
        compiler_params=pltpu.CompilerParams(
            dimension_semantics=("arbitrary",),
            vmem_limit_bytes=48 * 1024 * 1024,
        ),
        name="bdh_fused",
    )(embeddings, Dx, Dy, E,
      jnp.asarray(dm), jnp.asarray(lones), jnp.asarray(t0s))


# speedup vs baseline: 1.2634x; 1.2237x over previous
import numpy as np
import jax
import jax.numpy as jnp
from jax.experimental import pallas as pl
from jax.experimental.pallas import tpu as pltpu

_U_DECAY = 0.97
_X_DECAY = 0.97
_LN_EPS = 1e-5
_L1_EPS = 1e-12
_NEG = -1e30
_LNU = float(np.log(np.float64(_U_DECAY)))
_LNX = float(np.log(np.float64(_X_DECAY)))


def _ln(x):
    m = jnp.mean(x, axis=-1, keepdims=True)
    v = jnp.mean((x - m) ** 2, axis=-1, keepdims=True)
    return (x - m) * jax.lax.rsqrt(v + _LN_EPS)


def _body(packed_ref, out_ref, xu_ref, sc_ref, cc_ref):
    nbt, n = xu_ref.shape
    nb, t, d = out_ref.shape
    emb2 = packed_ref[0:nbt, :]
    emb = emb2.reshape(nb, t, d)
    dx = packed_ref[nbt:nbt + n, :]
    dy = packed_ref[nbt + n:nbt + 2 * n, :]
    et = packed_ref[nbt + 2 * n:nbt + 3 * n, :]

    xu_ref[...] = jnp.maximum(
        jax.lax.dot_general(
            emb2, dx, (((1,), (1,)), ((), ())),
            preferred_element_type=jnp.float32),
        0.0,
    )

    sc_ref[:, 0:nb] = jnp.concatenate(
        [jnp.sum(xu_ref[b * t:(b + 1) * t, :], axis=1, keepdims=True)
         for b in range(nb)],
        axis=1,
    )

    def sstep(k, sig):
        base = pl.multiple_of(k * 8, 8)
        tile = sc_ref[pl.ds(base, 8), :]
        cs = []
        for j in range(8):
            xi = _X_DECAY * sig + tile[j:j + 1, :]
            cs.append(jnp.maximum(xi, _L1_EPS))
            sig = jnp.minimum(xi * (1.0 / _L1_EPS), 1.0)
        cc_ref[pl.ds(base, 8), :] = jnp.concatenate(cs, axis=0)
        return sig

    jax.lax.fori_loop(0, t // 8, sstep, jnp.zeros((1, 128), jnp.float32),
                      unroll=2)

    ii = jax.lax.broadcasted_iota(jnp.int32, (t, t), 0)
    jj = jax.lax.broadcasted_iota(jnp.int32, (t, t), 1)
    lower = jj < ii
    fij = (ii - jj).astype(jnp.float32)
    dm = jnp.where(lower, jnp.exp(_LNU * (fij - 1.0)), 0.0)
    t0s = jnp.where(lower, _LNX * fij, _NEG)
    lones = jnp.where(jj <= ii, 1.0, 0.0)

    cl = jnp.log(cc_ref[...])
    lc = jnp.dot(lones, cl,
                 preferred_element_type=jnp.float32)
    lcs_t = jnp.transpose(lc - cl)
    a_list = []
    for b in range(nb):
        u = xu_ref[b * t:(b + 1) * t, :]
        w0 = jnp.exp(t0s + lcs_t[b:b + 1, :] - lc[:, b:b + 1])
        rcp = 1.0 / cc_ref[:, b:b + 1]
        x = u * rcp + jnp.dot(w0, u, preferred_element_type=jnp.float32)
        xu_ref[b * t:(b + 1) * t, :] = x

        g = jax.lax.dot_general(
            x, x, (((1,), (1,)), ((), ())),
            preferred_element_type=jnp.float32,
        )
        vl = _ln(emb[b])
        a_list.append(
            jnp.dot(dm * g, vl, preferred_element_type=jnp.float32))

    a_all = jnp.concatenate(a_list, axis=0)
    y = jnp.maximum(
        jax.lax.dot_general(
            _ln(a_all), dy, (((1,), (1,)), ((), ())),
            preferred_element_type=jnp.float32),
        0.0,
    ) * xu_ref[...]
    out_ref[...] = _ln(
        jnp.dot(y, et, preferred_element_type=jnp.float32)
    ).reshape(nb, t, d)


def kernel(embeddings, E, Dx, Dy):
    b, t, d = embeddings.shape
    n = E.shape[1]
    nb = 8

    packed = jnp.concatenate(
        [embeddings.reshape(b * t, d), Dx, Dy, jnp.transpose(E)], axis=0)

    return pl.pallas_call(
        _body,
        grid=(1,),
        in_specs=[
            pl.BlockSpec((b * t + 3 * n, d), lambda c: (0, 0)),
        ],
        out_specs=pl.BlockSpec((nb, t, d), lambda c: (c, 0, 0)),
        out_shape=jax.ShapeDtypeStruct((b, t, d), jnp.float32),
        scratch_shapes=[pltpu.VMEM((nb * t, n), jnp.float32)]
        + [pltpu.VMEM((t, 128), jnp.float32) for _ in range(2)],
        compiler_params=pltpu.CompilerParams(
            dimension_semantics=("arbitrary",),
            vmem_limit_bytes=48 * 1024 * 1024,
        ),
        name="bdh_fused",
    )(packed)
